# Initial kernel scaffold; baseline (speedup 1.0000x reference)
import jax
import jax.numpy as jnp
from jax.experimental import pallas as pl


def kernel(inp, emb0, emb1, emb2, proj0, proj1, proj2):
    raise NotImplementedError("write your pallas kernel here")


# speedup vs baseline: 52.4472x; 52.4472x over previous
import functools

import jax
import jax.numpy as jnp
from jax import lax
from jax.experimental import pallas as pl
from jax.experimental.pallas import tpu as pltpu
from jax.experimental.pallas import tpu_sc as plsc

N_TOKEN = 1000000
D_PROJ = 128
EMB_SCALE = float(D_PROJ) ** 0.5

ROW_BLOCK = 2000
NBLK0 = 100000 // ROW_BLOCK
NBLK1 = 400000 // ROW_BLOCK
NBLK2 = 500000 // ROW_BLOCK
NBLK = NBLK0 + NBLK1 + NBLK2

SC_NC = 2
SC_NS = 16
NW = SC_NC * SC_NS
CHUNK = 128


def _table_body(e0, e1, e2, p0, p1, p2, out):
    i = pl.program_id(0)
    dn = (((1,), (1,)), ((), ()))

    @pl.when(i < NBLK0)
    def _():
        out[...] = lax.dot_general(
            e0[...], p0[...], dn, preferred_element_type=jnp.float32
        ) * EMB_SCALE

    @pl.when(jnp.logical_and(i >= NBLK0, i < NBLK0 + NBLK1))
    def _():
        out[...] = lax.dot_general(
            e1[...], p1[...], dn, preferred_element_type=jnp.float32
        ) * EMB_SCALE

    @pl.when(i >= NBLK0 + NBLK1)
    def _():
        out[...] = lax.dot_general(
            e2[...], p2[...], dn, preferred_element_type=jnp.float32
        ) * EMB_SCALE


def _build_table(emb0, emb1, emb2, proj0, proj1, proj2):
    return pl.pallas_call(
        _table_body,
        grid=(NBLK,),
        in_specs=[
            pl.BlockSpec((ROW_BLOCK, 128),
                         lambda i: (jnp.minimum(i, NBLK0 - 1), 0)),
            pl.BlockSpec((ROW_BLOCK, 64),
                         lambda i: (jnp.clip(i - NBLK0, 0, NBLK1 - 1), 0)),
            pl.BlockSpec((ROW_BLOCK, 32),
                         lambda i: (jnp.clip(i - NBLK0 - NBLK1, 0, NBLK2 - 1), 0)),
            pl.BlockSpec((D_PROJ, 128), lambda i: (0, 0)),
            pl.BlockSpec((D_PROJ, 64), lambda i: (0, 0)),
            pl.BlockSpec((D_PROJ, 32), lambda i: (0, 0)),
        ],
        out_specs=pl.BlockSpec((ROW_BLOCK, D_PROJ), lambda i: (i, 0)),
        out_shape=jax.ShapeDtypeStruct((N_TOKEN, D_PROJ), jnp.float32),
    )(emb0, emb1, emb2, proj0, proj1, proj2)


def _make_gather(batch):
    b_per_w = batch // NW
    nchunk = b_per_w // CHUNK
    mesh = plsc.VectorSubcoreMesh(core_axis_name="c", subcore_axis_name="s")

    @functools.partial(
        pl.kernel,
        mesh=mesh,
        out_type=jax.ShapeDtypeStruct((batch, D_PROJ), jnp.float32),
        scratch_types=[
            pltpu.VMEM((CHUNK,), jnp.int32),
            pltpu.VMEM((CHUNK, D_PROJ), jnp.float32),
            pltpu.SemaphoreType.DMA,
        ],
    )
    def gather_kernel(table_hbm, idx_hbm, out_hbm, idx_v, rows_v, sem):
        wid = lax.axis_index("s") * SC_NC + lax.axis_index("c")
        base = wid * b_per_w

        def body(c, carry):
            off = base + c * CHUNK
            pltpu.sync_copy(idx_hbm.at[pl.ds(off, CHUNK)], idx_v)
            pltpu.async_copy(table_hbm.at[idx_v], rows_v, sem).wait()
            pltpu.sync_copy(rows_v, out_hbm.at[pl.ds(off, CHUNK)])
            return carry

        lax.fori_loop(0, nchunk, body, 0)

    return gather_kernel


def kernel(inp, emb0, emb1, emb2, proj0, proj1, proj2):
    table = _build_table(emb0, emb1, emb2, proj0, proj1, proj2)
    inp_flat = inp.reshape(-1)
    out_flat = _make_gather(inp_flat.shape[0])(table, inp_flat)
    return out_flat.reshape(inp.shape + (D_PROJ,))


# speedup vs baseline: 75.2891x; 1.4355x over previous
import functools

import jax
import jax.numpy as jnp
from jax import lax
from jax.experimental import pallas as pl
from jax.experimental.pallas import tpu as pltpu
from jax.experimental.pallas import tpu_sc as plsc

N_TOKEN = 1000000
D_PROJ = 128
EMB_SCALE = float(D_PROJ) ** 0.5

ROW_BLOCK = 4000
NBLK0 = 100000 // ROW_BLOCK
NBLK1 = 400000 // ROW_BLOCK
NBLK2 = 500000 // ROW_BLOCK
NBLK = NBLK0 + NBLK1 + NBLK2

SC_NC = 2
SC_NS = 16
NW = SC_NC * SC_NS
CHUNK = 128
NBUF = 4


def _table_body(e0, e1, e2, p0, p1, p2, out):
    i = pl.program_id(0)
    dn = (((1,), (0,)), ((), ()))

    @pl.when(i < NBLK0)
    def _():
        out[...] = lax.dot_general(
            e0[...].astype(jnp.bfloat16), p0[...].astype(jnp.bfloat16),
            dn, preferred_element_type=jnp.float32)

    @pl.when(jnp.logical_and(i >= NBLK0, i < NBLK0 + NBLK1))
    def _():
        out[...] = lax.dot_general(
            e1[...].astype(jnp.bfloat16), p1[...].astype(jnp.bfloat16),
            dn, preferred_element_type=jnp.float32)

    @pl.when(i >= NBLK0 + NBLK1)
    def _():
        out[...] = lax.dot_general(
            e2[...].astype(jnp.bfloat16), p2[...].astype(jnp.bfloat16),
            dn, preferred_element_type=jnp.float32)


def _build_table(emb0, emb1, emb2, p0t, p1t, p2t):
    return pl.pallas_call(
        _table_body,
        grid=(NBLK,),
        in_specs=[
            pl.BlockSpec((ROW_BLOCK, 128),
                         lambda i: (jnp.minimum(i, NBLK0 - 1), 0)),
            pl.BlockSpec((ROW_BLOCK, 64),
                         lambda i: (jnp.clip(i - NBLK0, 0, NBLK1 - 1), 0)),
            pl.BlockSpec((ROW_BLOCK, 32),
                         lambda i: (jnp.clip(i - NBLK0 - NBLK1, 0, NBLK2 - 1), 0)),
            pl.BlockSpec((128, D_PROJ), lambda i: (0, 0)),
            pl.BlockSpec((64, D_PROJ), lambda i: (0, 0)),
            pl.BlockSpec((32, D_PROJ), lambda i: (0, 0)),
        ],
        out_specs=pl.BlockSpec((ROW_BLOCK, D_PROJ), lambda i: (i, 0)),
        out_shape=jax.ShapeDtypeStruct((N_TOKEN, D_PROJ), jnp.float32),
    )(emb0, emb1, emb2, p0t, p1t, p2t)


def _make_gather(batch):
    b_per_w = batch // NW
    nchunk = b_per_w // CHUNK
    assert batch == NW * nchunk * CHUNK and nchunk % NBUF == 0
    mesh = plsc.VectorSubcoreMesh(core_axis_name="c", subcore_axis_name="s")

    @functools.partial(
        pl.kernel,
        mesh=mesh,
        out_type=jax.ShapeDtypeStruct((batch, D_PROJ), jnp.float32),
        scratch_types=[
            pltpu.VMEM((nchunk, CHUNK), jnp.int32),
            pltpu.VMEM((NBUF, CHUNK, D_PROJ), jnp.float32),
        ] + [pltpu.SemaphoreType.DMA] * (2 * NBUF),
    )
    def gather_kernel(table_hbm, idx_hbm, out_hbm, idx2, rows, *sems):
        gsem = sems[:NBUF]
        ssem = sems[NBUF:]
        wid = lax.axis_index("s") * SC_NC + lax.axis_index("c")
        base = wid * b_per_w

        pltpu.sync_copy(idx_hbm.at[wid], idx2)

        def g_copy(c, b, sem):
            return pltpu.make_async_copy(
                table_hbm.at[idx2.at[c]], rows.at[b], sem)

        def s_copy(c, b, sem):
            return pltpu.make_async_copy(
                rows.at[b], out_hbm.at[pl.ds(base + c * CHUNK, CHUNK)], sem)

        for b in range(NBUF - 1):
            g_copy(b, b, gsem[b]).start()

        def outer(t, carry):
            for b in range(NBUF):
                c = t * NBUF + b
                fb = (b - 1) % NBUF
                f = c + NBUF - 1

                @pl.when(jnp.logical_and(c >= 1, f < nchunk))
                def _():
                    s_copy(c - 1, fb, ssem[fb]).wait()

                @pl.when(f < nchunk)
                def _():
                    g_copy(f, fb, gsem[fb]).start()

                g_copy(c, b, gsem[b]).wait()
                s_copy(c, b, ssem[b]).start()
            return carry

        lax.fori_loop(0, nchunk // NBUF, outer, 0)

        for b in range(NBUF):
            s_copy(nchunk - NBUF + b, b, ssem[b]).wait()

    return gather_kernel


def kernel(inp, emb0, emb1, emb2, proj0, proj1, proj2):
    p0t = proj0.T * EMB_SCALE
    p1t = proj1.T * EMB_SCALE
    p2t = proj2.T * EMB_SCALE
    table = _build_table(emb0, emb1, emb2, p0t, p1t, p2t)
    batch = inp.size
    idx3 = inp.reshape(NW, batch // (NW * CHUNK), CHUNK)
    out_flat = _make_gather(batch)(table, idx3)
    return out_flat.reshape(inp.shape + (D_PROJ,))


# speedup vs baseline: 79.0589x; 1.0501x over previous
import functools

import jax
import jax.numpy as jnp
from jax import lax
from jax.experimental import pallas as pl
from jax.experimental.pallas import tpu as pltpu
from jax.experimental.pallas import tpu_sc as plsc

N_TOKEN = 1000000
D_PROJ = 128
EMB_SCALE = float(D_PROJ) ** 0.5

ROW_BLOCK = 10000
NBLK0 = 100000 // ROW_BLOCK
NBLK1 = 400000 // ROW_BLOCK
NBLK2 = 500000 // ROW_BLOCK
NBLK = NBLK0 + NBLK1 + NBLK2

SC_NC = 2
SC_NS = 16
NW = SC_NC * SC_NS
CHUNK = 128
NBUF = 4


def _table_body(e0, e1, e2, p0, p1, p2, out):
    i = pl.program_id(0)
    dn = (((1,), (0,)), ((), ()))

    @pl.when(i < NBLK0)
    def _():
        out[...] = lax.dot_general(
            e0[...].astype(jnp.bfloat16), p0[...].astype(jnp.bfloat16),
            dn, preferred_element_type=jnp.float32)

    @pl.when(jnp.logical_and(i >= NBLK0, i < NBLK0 + NBLK1))
    def _():
        out[...] = lax.dot_general(
            e1[...].astype(jnp.bfloat16), p1[...].astype(jnp.bfloat16),
            dn, preferred_element_type=jnp.float32)

    @pl.when(i >= NBLK0 + NBLK1)
    def _():
        out[...] = lax.dot_general(
            e2[...].astype(jnp.bfloat16), p2[...].astype(jnp.bfloat16),
            dn, preferred_element_type=jnp.float32)


def _build_table(emb0, emb1, emb2, p0t, p1t, p2t):
    return pl.pallas_call(
        _table_body,
        grid=(NBLK,),
        in_specs=[
            pl.BlockSpec((ROW_BLOCK, 128),
                         lambda i: (jnp.minimum(i, NBLK0 - 1), 0)),
            pl.BlockSpec((ROW_BLOCK, 64),
                         lambda i: (jnp.clip(i - NBLK0, 0, NBLK1 - 1), 0)),
            pl.BlockSpec((ROW_BLOCK, 32),
                         lambda i: (jnp.clip(i - NBLK0 - NBLK1, 0, NBLK2 - 1), 0)),
            pl.BlockSpec((128, D_PROJ), lambda i: (0, 0)),
            pl.BlockSpec((64, D_PROJ), lambda i: (0, 0)),
            pl.BlockSpec((32, D_PROJ), lambda i: (0, 0)),
        ],
        out_specs=pl.BlockSpec((ROW_BLOCK, D_PROJ), lambda i: (i, 0)),
        out_shape=jax.ShapeDtypeStruct((N_TOKEN, D_PROJ), jnp.float32),
    )(emb0, emb1, emb2, p0t, p1t, p2t)


def _make_gather(batch):
    b_per_w = batch // NW
    nchunk = b_per_w // CHUNK
    assert batch == NW * nchunk * CHUNK and nchunk % NBUF == 0
    mesh = plsc.VectorSubcoreMesh(core_axis_name="c", subcore_axis_name="s")

    @functools.partial(
        pl.kernel,
        mesh=mesh,
        out_type=jax.ShapeDtypeStruct((batch, D_PROJ), jnp.float32),
        scratch_types=[
            pltpu.VMEM((nchunk, CHUNK), jnp.int32),
            pltpu.VMEM((NBUF, CHUNK, D_PROJ), jnp.float32),
        ] + [pltpu.SemaphoreType.DMA] * (2 * NBUF),
    )
    def gather_kernel(table_hbm, idx_hbm, out_hbm, idx2, rows, *sems):
        gsem = sems[:NBUF]
        ssem = sems[NBUF:]
        wid = lax.axis_index("s") * SC_NC + lax.axis_index("c")
        base = wid * b_per_w

        pltpu.sync_copy(idx_hbm.at[wid], idx2)

        def g_copy(c, b, sem):
            return pltpu.make_async_copy(
                table_hbm.at[idx2.at[c]], rows.at[b], sem)

        def s_copy(c, b, sem):
            return pltpu.make_async_copy(
                rows.at[b], out_hbm.at[pl.ds(base + c * CHUNK, CHUNK)], sem)

        for b in range(NBUF - 1):
            g_copy(b, b, gsem[b]).start()

        def outer(t, carry):
            for b in range(NBUF):
                c = t * NBUF + b
                fb = (b - 1) % NBUF
                f = c + NBUF - 1

                @pl.when(jnp.logical_and(c >= 1, f < nchunk))
                def _():
                    s_copy(c - 1, fb, ssem[fb]).wait()

                @pl.when(f < nchunk)
                def _():
                    g_copy(f, fb, gsem[fb]).start()

                g_copy(c, b, gsem[b]).wait()
                s_copy(c, b, ssem[b]).start()
            return carry

        lax.fori_loop(0, nchunk // NBUF, outer, 0)

        for b in range(NBUF):
            s_copy(nchunk - NBUF + b, b, ssem[b]).wait()

    return gather_kernel


def kernel(inp, emb0, emb1, emb2, proj0, proj1, proj2):
    p0t = proj0.T * EMB_SCALE
    p1t = proj1.T * EMB_SCALE
    p2t = proj2.T * EMB_SCALE
    table = _build_table(emb0, emb1, emb2, p0t, p1t, p2t)
    batch = inp.size
    idx3 = inp.reshape(NW, batch // (NW * CHUNK), CHUNK)
    out_flat = _make_gather(batch)(table, idx3)
    return out_flat.reshape(inp.shape + (D_PROJ,))


# speedup vs baseline: 79.2282x; 1.0021x over previous
import functools

import jax
import jax.numpy as jnp
from jax import lax
from jax.experimental import pallas as pl
from jax.experimental.pallas import tpu as pltpu
from jax.experimental.pallas import tpu_sc as plsc

N_TOKEN = 1000000
D_PROJ = 128
EMB_SCALE = float(D_PROJ) ** 0.5

ROW_BLOCK = 10000
NBLK0 = 100000 // ROW_BLOCK
NBLK1 = 400000 // ROW_BLOCK
NBLK2 = 500000 // ROW_BLOCK
NBLK = NBLK0 + NBLK1 + NBLK2

SC_NC = 2
SC_NS = 16
NW = SC_NC * SC_NS
CHUNK = 128
NBUF = 4


SBUF = 4


def _table_body(e0, e1, e2, p0, p1, p2, out_hbm, obuf, osem):
    i = pl.program_id(0)
    dn = (((1,), (0,)), ((), ()))

    def out_copy(blk, b):
        return pltpu.make_async_copy(
            obuf.at[b],
            out_hbm.at[pl.ds(blk * ROW_BLOCK, ROW_BLOCK)],
            osem.at[b])

    b = lax.rem(i, SBUF)

    @pl.when(i >= SBUF)
    def _():
        out_copy(i - SBUF, b).wait()

    @pl.when(i < NBLK0)
    def _():
        obuf[b] = lax.dot_general(
            e0[...].astype(jnp.bfloat16), p0[...].astype(jnp.bfloat16),
            dn, preferred_element_type=jnp.float32)

    @pl.when(jnp.logical_and(i >= NBLK0, i < NBLK0 + NBLK1))
    def _():
        obuf[b] = lax.dot_general(
            e1[...].astype(jnp.bfloat16), p1[...].astype(jnp.bfloat16),
            dn, preferred_element_type=jnp.float32)

    @pl.when(i >= NBLK0 + NBLK1)
    def _():
        obuf[b] = lax.dot_general(
            e2[...].astype(jnp.bfloat16), p2[...].astype(jnp.bfloat16),
            dn, preferred_element_type=jnp.float32)

    out_copy(i, b).start()

    @pl.when(i == NBLK - 1)
    def _():
        for j in range(SBUF):
            out_copy(i - j, (b - j) % SBUF).wait()


def _build_table(emb0, emb1, emb2, p0t, p1t, p2t):
    return pl.pallas_call(
        _table_body,
        grid=(NBLK,),
        in_specs=[
            pl.BlockSpec((ROW_BLOCK, 128),
                         lambda i: (jnp.minimum(i, NBLK0 - 1), 0)),
            pl.BlockSpec((ROW_BLOCK, 64),
                         lambda i: (jnp.clip(i - NBLK0, 0, NBLK1 - 1), 0)),
            pl.BlockSpec((ROW_BLOCK, 32),
                         lambda i: (jnp.clip(i - NBLK0 - NBLK1, 0, NBLK2 - 1), 0)),
            pl.BlockSpec((128, D_PROJ), lambda i: (0, 0)),
            pl.BlockSpec((64, D_PROJ), lambda i: (0, 0)),
            pl.BlockSpec((32, D_PROJ), lambda i: (0, 0)),
        ],
        scratch_shapes=[
            pltpu.VMEM((SBUF, ROW_BLOCK, D_PROJ), jnp.float32),
            pltpu.SemaphoreType.DMA((SBUF,)),
        ],
        out_specs=pl.BlockSpec(memory_space=pl.ANY),
        out_shape=jax.ShapeDtypeStruct((N_TOKEN, D_PROJ), jnp.float32),
    )(emb0, emb1, emb2, p0t, p1t, p2t)


def _make_gather(batch):
    b_per_w = batch // NW
    nchunk = b_per_w // CHUNK
    assert batch == NW * nchunk * CHUNK and nchunk % NBUF == 0
    mesh = plsc.VectorSubcoreMesh(core_axis_name="c", subcore_axis_name="s")

    @functools.partial(
        pl.kernel,
        mesh=mesh,
        out_type=jax.ShapeDtypeStruct((batch, D_PROJ), jnp.float32),
        scratch_types=[
            pltpu.VMEM((nchunk, CHUNK), jnp.int32),
            pltpu.VMEM((NBUF, CHUNK, D_PROJ), jnp.float32),
        ] + [pltpu.SemaphoreType.DMA] * (2 * NBUF),
    )
    def gather_kernel(table_hbm, idx_hbm, out_hbm, idx2, rows, *sems):
        gsem = sems[:NBUF]
        ssem = sems[NBUF:]
        wid = lax.axis_index("s") * SC_NC + lax.axis_index("c")
        base = wid * b_per_w

        pltpu.sync_copy(idx_hbm.at[wid], idx2)

        def g_copy(c, b, sem):
            return pltpu.make_async_copy(
                table_hbm.at[idx2.at[c]], rows.at[b], sem)

        def s_copy(c, b, sem):
            return pltpu.make_async_copy(
                rows.at[b], out_hbm.at[pl.ds(base + c * CHUNK, CHUNK)], sem)

        for b in range(NBUF - 1):
            g_copy(b, b, gsem[b]).start()

        def outer(t, carry):
            for b in range(NBUF):
                c = t * NBUF + b
                fb = (b - 1) % NBUF
                f = c + NBUF - 1

                @pl.when(jnp.logical_and(c >= 1, f < nchunk))
                def _():
                    s_copy(c - 1, fb, ssem[fb]).wait()

                @pl.when(f < nchunk)
                def _():
                    g_copy(f, fb, gsem[fb]).start()

                g_copy(c, b, gsem[b]).wait()
                s_copy(c, b, ssem[b]).start()
            return carry

        lax.fori_loop(0, nchunk // NBUF, outer, 0)

        for b in range(NBUF):
            s_copy(nchunk - NBUF + b, b, ssem[b]).wait()

    return gather_kernel


def kernel(inp, emb0, emb1, emb2, proj0, proj1, proj2):
    p0t = proj0.T * EMB_SCALE
    p1t = proj1.T * EMB_SCALE
    p2t = proj2.T * EMB_SCALE
    table = _build_table(emb0, emb1, emb2, p0t, p1t, p2t)
    batch = inp.size
    idx3 = inp.reshape(NW, batch // (NW * CHUNK), CHUNK)
    out_flat = _make_gather(batch)(table, idx3)
    return out_flat.reshape(inp.shape + (D_PROJ,))
